# baseline (speedup 1.0000x reference)
import functools

import jax
import jax.numpy as jnp
from jax import lax
from jax.experimental import pallas as pl
from jax.experimental.pallas import tpu as pltpu
from jax.experimental.pallas import tpu_sc as plsc

_NC, _NS = 2, 16
_NW = _NC * _NS
_CHUNK = 16 * 1024
_LANES = 16


_ROWS = 8
_D = 1024
_B = 4


def _sc_body(x_hbm, p_hbm, o_hbm, xb, pb, sx, sp, so, *, s_full, s_cover):
    spw = s_cover // _NW
    nstep = spw // _ROWS
    wid = lax.axis_index("s") * _NC + lax.axis_index("c")
    s_base = wid * spw

    def pos_copy(step, slot):
        s0 = s_base + step * _ROWS
        return pltpu.make_async_copy(
            p_hbm.at[pl.ds(s0, _ROWS)], pb.at[slot], sp.at[slot])

    def x_copy(step, slot, b):
        s0 = s_base + step * _ROWS
        return pltpu.make_async_copy(
            x_hbm.at[pl.ds(b * s_full + s0, _ROWS)], xb.at[slot, b], sx.at[slot, b])

    def out_copy(step, slot, b):
        s0 = s_base + step * _ROWS
        return pltpu.make_async_copy(
            xb.at[slot, b], o_hbm.at[pl.ds(b * s_cover + s0, _ROWS)], so.at[slot, b])

    def compute_batch(slot, b):
        @plsc.parallel_loop(0, _ROWS * _D, step=_LANES, unroll=8)
        def _(off):
            r = off // _D
            c = off % _D
            pv = pb[slot, r, pl.ds(c, _LANES)]
            plsc.addupdate(xb.at[slot, b, r, pl.ds(c, _LANES)], pv)

    for s0 in (0, 1):
        pos_copy(s0, s0).start()
        for b in range(_B):
            x_copy(s0, s0, b).start()
    for s in range(nstep):
        slot = s % 3
        if s + 2 < nstep:
            nslot = (s + 2) % 3
            pos_copy(s + 2, nslot).start()
            for b in range(_B):
                if s >= 1:
                    out_copy(s - 1, nslot, b).wait()
                x_copy(s + 2, nslot, b).start()
        pos_copy(s, slot).wait()
        for b in range(_B):
            x_copy(s, slot, b).wait()
            compute_batch(slot, b)
            out_copy(s, slot, b).start()
    for s in (nstep - 3, nstep - 2, nstep - 1):
        if s >= 0:
            for b in range(_B):
                out_copy(s, s % 3, b).wait()


def _sc_add(x, pos_table, s_cover=None):
    B, S, D = x.shape
    if s_cover is None:
        s_cover = S
    xf = x.reshape(B * S, D)
    pf = pos_table
    run = pl.kernel(
        functools.partial(_sc_body, s_full=S, s_cover=s_cover),
        out_type=jax.ShapeDtypeStruct((B * s_cover, D), x.dtype),
        mesh=plsc.VectorSubcoreMesh(
            core_axis_name="c", subcore_axis_name="s",
            num_cores=_NC, num_subcores=_NS,
        ),
        scratch_types=[
            pltpu.VMEM((3, _B, _ROWS, _D), jnp.float32),
            pltpu.VMEM((3, _ROWS, _D), jnp.float32),
            pltpu.SemaphoreType.DMA((3, _B)),
            pltpu.SemaphoreType.DMA((3,)),
            pltpu.SemaphoreType.DMA((3, _B)),
        ],
    )
    return run(xf, pf).reshape(B, s_cover, D)


def kernel(x, pos_table):
    return _sc_add(x, pos_table)
